# baseline (speedup 1.0000x reference)
import jax
import jax.numpy as jnp
from jax import lax
from jax.experimental import pallas as pl
from jax.experimental.pallas import tpu as pltpu
from jax.experimental.pallas import tpu_sc as plsc

N = 100000
D = 128
MAX_DEG = 10
C0 = 10000
CD = 9000

NC = 2
NS = 16
NW = NC * NS

CHUNK_ROWS = {1: 72, 2: 72, 3: 72, 4: 40, 5: 40, 6: 40,
              7: 24, 8: 24, 9: 24, 10: 24}

G0 = 80
NCHUNK0 = C0 // G0
TRIPS0 = (NCHUNK0 + NW - 1) // NW

_mesh = plsc.VectorSubcoreMesh(
    core_axis_name="c", subcore_axis_name="s", num_cores=NC, num_subcores=NS
)


def _body(atoms, adjcat, out, gsem0, gsem1, ssem0, ssem1, osem0, osem1):
    wid = lax.axis_index("s") * NC + lax.axis_index("c")
    gsems = [gsem0, gsem1]
    ssems = [ssem0, ssem1]
    osems = [osem0, osem1]

    def deg0(cb):
        def copy_body(t, carry):
            chunk = wid + NW * t

            @pl.when(chunk < NCHUNK0)
            def _():
                base = chunk * G0
                pltpu.sync_copy(atoms.at[pl.ds(base, G0)], cb)
                pltpu.sync_copy(cb, out.at[pl.ds(base, G0)])

            return carry

        lax.fori_loop(0, TRIPS0, copy_body, 0)

    pl.run_scoped(deg0, pltpu.VMEM((G0, D), jnp.float32))

    for d in range(1, MAX_DEG + 1):
        ioff = CD * (d * (d - 1) // 2)
        row0 = C0 + (d - 1) * CD
        g = CHUNK_ROWS[d]
        nchunk = CD // g
        trips = (nchunk + NW - 1) // NW

        def degree(ib0, ib1, gb0, gb1, sb0, sb1, ob0, ob1,
                   ioff=ioff, row0=row0, g=g, nchunk=nchunk, trips=trips, d=d):
            ibs = [ib0, ib1]
            gbs, sbs, obs = [gb0, gb1], [sb0, sb1], [ob0, ob1]

            def start(t, p):
                chunk = wid + NW * t

                @pl.when(chunk < nchunk)
                def _():
                    rbase = row0 + chunk * g
                    pltpu.sync_copy(
                        adjcat.at[pl.ds(ioff + chunk * g * d, g * d)], ibs[p])
                    pltpu.async_copy(atoms.at[ibs[p]], gbs[p], gsems[p])
                    pltpu.async_copy(atoms.at[pl.ds(rbase, g)], sbs[p], ssems[p])

            def finish(t, u, p):
                chunk = wid + NW * t

                @pl.when(chunk < nchunk)
                def _():
                    rbase = row0 + chunk * g
                    pltpu.make_async_copy(atoms.at[ibs[p]], gbs[p],
                                          gsems[p]).wait()
                    pltpu.make_async_copy(atoms.at[pl.ds(rbase, g)], sbs[p],
                                          ssems[p]).wait()

                    @pl.when(u >= 1)
                    def _():
                        pltpu.make_async_copy(obs[p], out.at[pl.ds(row0, g)],
                                              osems[p]).wait()

                    def row_body(i, c2):
                        for c in range(D // 16):
                            sl = pl.ds(c * 16, 16)
                            v = sbs[p][i, sl]
                            for j in range(d):
                                v = jnp.maximum(v, gbs[p][i * d + j, sl])
                            obs[p][i, sl] = v
                        return c2

                    lax.fori_loop(0, g, row_body, 0)
                    pltpu.async_copy(obs[p], out.at[pl.ds(rbase, g)], osems[p])

            start(0, 0)
            start(1, 1)

            def pair_body(u, carry):
                t0 = 2 * u
                finish(t0, u, 0)
                start(t0 + 2, 0)
                finish(t0 + 1, u, 1)
                start(t0 + 3, 1)
                return carry

            lax.fori_loop(0, trips // 2, pair_body, 0)

            @pl.when(wid + NW * (trips - 2) < nchunk)
            def _():
                pltpu.make_async_copy(obs[0], out.at[pl.ds(row0, g)],
                                      osems[0]).wait()

            @pl.when(wid + NW * (trips - 1) < nchunk)
            def _():
                pltpu.make_async_copy(obs[1], out.at[pl.ds(row0, g)],
                                      osems[1]).wait()

        pl.run_scoped(
            degree,
            pltpu.VMEM((g * d,), jnp.int32),
            pltpu.VMEM((g * d,), jnp.int32),
            pltpu.VMEM((g * d, D), jnp.float32),
            pltpu.VMEM((g * d, D), jnp.float32),
            pltpu.VMEM((g, D), jnp.float32),
            pltpu.VMEM((g, D), jnp.float32),
            pltpu.VMEM((g, D), jnp.float32),
            pltpu.VMEM((g, D), jnp.float32),
        )


_pool = pl.kernel(
    _body,
    out_type=jax.ShapeDtypeStruct((N, D), jnp.float32),
    mesh=_mesh,
    scratch_types=[
        pltpu.SemaphoreType.DMA,
        pltpu.SemaphoreType.DMA,
        pltpu.SemaphoreType.DMA,
        pltpu.SemaphoreType.DMA,
        pltpu.SemaphoreType.DMA,
        pltpu.SemaphoreType.DMA,
    ],
)


def kernel(atoms, deg_slice, membership, deg_adj_1, deg_adj_2, deg_adj_3,
           deg_adj_4, deg_adj_5, deg_adj_6, deg_adj_7, deg_adj_8,
           deg_adj_9, deg_adj_10):
    adjcat = jnp.concatenate([
        a.reshape(-1) for a in
        (deg_adj_1, deg_adj_2, deg_adj_3, deg_adj_4, deg_adj_5,
         deg_adj_6, deg_adj_7, deg_adj_8, deg_adj_9, deg_adj_10)])
    return _pool(atoms, adjcat)
